# baseline (speedup 1.0000x reference)
import functools

import jax
import jax.numpy as jnp
from jax import lax
from jax.experimental import pallas as pl
from jax.experimental.pallas import tpu as pltpu
from jax.experimental.pallas import tpu_sc as plsc

_NC, _NS, _L = 2, 16, 16
_NW = _NC * _NS
_CH = 16
_NBUF_X = 4
_NBUF_R = 3
_DEPTH = 2


def _sc_gather_add(x_nc, idx_n, table):
    N, C = x_nc.shape
    n_per_w = N // _NW
    CH, NBX, NBR, D = _CH, _NBUF_X, _NBUF_R, _DEPTH
    n_ch = n_per_w // CH
    mesh = plsc.VectorSubcoreMesh(core_axis_name="c", subcore_axis_name="s")

    scratch = [
        pltpu.VMEM((n_per_w,), jnp.int32),
        pltpu.VMEM((NBX, CH, C), jnp.float32),
        pltpu.VMEM((NBR, CH, C), jnp.float32),
        pltpu.SemaphoreType.DMA((NBX,)),
        pltpu.SemaphoreType.DMA((NBR,)),
        pltpu.SemaphoreType.DMA((NBX,)),
    ]

    @functools.partial(
        pl.kernel,
        out_type=jax.ShapeDtypeStruct((N, C), jnp.float32),
        mesh=mesh,
        scratch_types=scratch,
    )
    def k(x_hbm, idx_hbm, tab_hbm, out_hbm, idx_v, xb, rb, sx, sr, so):
        wid = lax.axis_index("s") * _NC + lax.axis_index("c")
        base = wid * n_per_w
        pltpu.sync_copy(idx_hbm.at[pl.ds(base, n_per_w)], idx_v)

        def issue_in(ci, bx, br):
            off = base + ci * CH
            pltpu.async_copy(x_hbm.at[pl.ds(off, CH)], xb.at[bx], sx.at[bx])
            pltpu.async_copy(tab_hbm.at[idx_v.at[pl.ds(ci * CH, CH)]], rb.at[br],
                             sr.at[br])

        def wait_in(ci, bx, br):
            off = base + ci * CH
            pltpu.make_async_copy(x_hbm.at[pl.ds(off, CH)], xb.at[bx],
                                  sx.at[bx]).wait()
            pltpu.make_async_copy(
                tab_hbm.at[idx_v.at[pl.ds(ci * CH, CH)]], rb.at[br],
                sr.at[br]).wait()

        def issue_out(ci, bx):
            off = base + ci * CH
            pltpu.async_copy(xb.at[bx], out_hbm.at[pl.ds(off, CH)], so.at[bx])

        def wait_out(ci, bx):
            off = base + ci * CH
            pltpu.make_async_copy(xb.at[bx], out_hbm.at[pl.ds(off, CH)],
                                  so.at[bx]).wait()

        def add_rows(bx, br):
            def row(i, c2):
                for j in range(C // _L):
                    sl = pl.ds(j * _L, _L)
                    plsc.addupdate(xb.at[bx, i, sl], rb[br, i, sl])
                return c2

            lax.fori_loop(0, CH, row, 0)

        for p in range(D):
            issue_in(p, p % NBX, p % NBR)

        def body(ci, carry):
            bx = lax.rem(ci, NBX)
            br = lax.rem(ci, NBR)
            wait_in(ci, bx, br)
            nci = ci + D
            nbx = lax.rem(nci, NBX)
            nbr = lax.rem(nci, NBR)

            @pl.when(nci < n_ch)
            def _():
                @pl.when(nci >= NBX)
                def _():
                    wait_out(nci - NBX, nbx)

                issue_in(nci, nbx, nbr)

            add_rows(bx, br)
            issue_out(ci, bx)
            return carry

        lax.fori_loop(0, n_ch, body, 0)
        for t in range(NBX):
            ci = n_ch - NBX + t
            wait_out(ci, ci % NBX)

    return k(x_nc, idx_n, table)


def kernel(x_btc, times_bt, embeddings_tc, offset):
    B, T, C = x_btc.shape
    x = x_btc.reshape(B * T, C)
    idx = times_bt.reshape(B * T).astype(jnp.int32)
    out = _sc_gather_add(x, idx, embeddings_tc)
    return out.reshape(B, T, C)


# speedup vs baseline: 1.5975x; 1.5975x over previous
import functools

import jax
import jax.numpy as jnp
from jax import lax
from jax.experimental import pallas as pl
from jax.experimental.pallas import tpu as pltpu
from jax.experimental.pallas import tpu_sc as plsc

_NC, _NS, _L = 2, 16, 16
_NW = _NC * _NS
_CH = 16
_NBUF_X = 3
_NBUF_R = 3
_DEPTH = 2


def _sc_gather_add(x_nc, idx_n, table):
    N, C = x_nc.shape
    n_per_w = N // _NW
    CH, NBX, NBR, D = _CH, _NBUF_X, _NBUF_R, _DEPTH
    n_ch = n_per_w // CH
    mesh = plsc.VectorSubcoreMesh(core_axis_name="c", subcore_axis_name="s")

    scratch = [
        pltpu.VMEM((n_per_w,), jnp.int32),
        pltpu.VMEM((NBX, CH, C), jnp.float32),
        pltpu.VMEM((NBR, CH, C), jnp.float32),
        pltpu.SemaphoreType.DMA((NBX,)),
        pltpu.SemaphoreType.DMA((NBR,)),
        pltpu.SemaphoreType.DMA((NBX,)),
    ]

    @functools.partial(
        pl.kernel,
        out_type=jax.ShapeDtypeStruct((N, C), jnp.float32),
        mesh=mesh,
        scratch_types=scratch,
    )
    def k(x_hbm, idx_hbm, tab_hbm, out_hbm, idx_v, xb, rb, sx, sr, so):
        wid = lax.axis_index("s") * _NC + lax.axis_index("c")
        base = wid * n_per_w
        pltpu.sync_copy(idx_hbm.at[pl.ds(base, n_per_w)], idx_v)

        def issue_in(ci, bx, br):
            off = base + ci * CH
            pltpu.async_copy(x_hbm.at[pl.ds(off, CH)], xb.at[bx], sx.at[bx])
            pltpu.async_copy(tab_hbm.at[idx_v.at[pl.ds(ci * CH, CH)]], rb.at[br],
                             sr.at[br])

        def wait_in(ci, bx, br):
            off = base + ci * CH
            pltpu.make_async_copy(x_hbm.at[pl.ds(off, CH)], xb.at[bx],
                                  sx.at[bx]).wait()
            pltpu.make_async_copy(
                tab_hbm.at[idx_v.at[pl.ds(ci * CH, CH)]], rb.at[br],
                sr.at[br]).wait()

        def issue_out(ci, bx):
            off = base + ci * CH
            pltpu.async_copy(xb.at[bx], out_hbm.at[pl.ds(off, CH)], so.at[bx])

        def wait_out(ci, bx):
            off = base + ci * CH
            pltpu.make_async_copy(xb.at[bx], out_hbm.at[pl.ds(off, CH)],
                                  so.at[bx]).wait()

        def add_rows(bx, br):
            def row(i, c2):
                for j in range(C // _L):
                    sl = pl.ds(j * _L, _L)
                    plsc.addupdate(xb.at[bx, i, sl], rb[br, i, sl])
                return c2

            lax.fori_loop(0, CH, row, 0)

        for p in range(D):
            issue_in(p, p % NBX, p % NBR)

        def body(ci, carry):
            bx = lax.rem(ci, NBX)
            br = lax.rem(ci, NBR)
            wait_in(ci, bx, br)
            nci = ci + D
            nbx = lax.rem(nci, NBX)
            nbr = lax.rem(nci, NBR)

            @pl.when(nci < n_ch)
            def _():
                @pl.when(nci >= NBX)
                def _():
                    wait_out(nci - NBX, nbx)

                issue_in(nci, nbx, nbr)

            add_rows(bx, br)
            issue_out(ci, bx)
            return carry

        lax.fori_loop(0, n_ch, body, 0)
        for t in range(NBX):
            ci = n_ch - NBX + t
            wait_out(ci, ci % NBX)

    return k(x_nc, idx_n, table)


def kernel(x_btc, times_bt, embeddings_tc, offset):
    B, T, C = x_btc.shape
    x = x_btc.reshape(B * T, C)
    idx = times_bt.reshape(B * T).astype(jnp.int32)
    out = _sc_gather_add(x, idx, embeddings_tc)
    return out.reshape(B, T, C)
